# baseline (speedup 1.0000x reference)
import functools

import jax
import jax.numpy as jnp
from jax import lax
from jax.experimental import pallas as pl
from jax.experimental.pallas import tpu as pltpu
from jax.experimental.pallas import tpu_sc as plsc

_V = 100000
_D = 64
_SEQ = 16384
_NC, _NS = 2, 16
_NW = _NC * _NS
_FULL_TCOLS = 781
_TAIL_BASE = _FULL_TCOLS * 128
_CK = 4
_NFULL = 7
_WG = 6
_TRASH = _SEQ
_OUT_ROWS = _SEQ + 128
_CW = _CK * 128

_mesh = plsc.VectorSubcoreMesh(core_axis_name="c", subcore_axis_name="s")


@functools.partial(
    pl.kernel,
    mesh=_mesh,
    out_type=jax.ShapeDtypeStruct((_OUT_ROWS, 128), jnp.float32),
    scratch_types=[
        pltpu.VMEM((_SEQ,), jnp.int32),
        pltpu.VMEM((_SEQ,), jnp.int32),
        pltpu.VMEM((128, _CW), jnp.float32),
        pltpu.VMEM((2 * _WG * 16, 128), jnp.float32),
        pltpu.SemaphoreType.DMA,
        pltpu.SemaphoreType.DMA,
    ],
    compiler_params=pltpu.CompilerParams(needs_layout_passes=False),
)
def _emb(idx_hbm, wt_hbm, wtail_hbm, out_hbm,
         sub_v, pk_v, wbuf, rows_v, dsem, ssem):
    idx_v = sub_v
    wid = lax.axis_index("s") * _NC + lax.axis_index("c")
    lanes = lax.iota(jnp.int32, 16)
    c0_w = 24 * wid + jnp.minimum(wid, 13)
    ncols = 24 + (wid < 13).astype(jnp.int32)
    v_lo = c0_w * 128
    v_hi = jnp.where(wid == _NW - 1, _V, (c0_w + ncols) * 128)

    def fire_full(k):
        rb = (k % 2) * _D
        cdma = jnp.minimum(c0_w + _CK * k, _FULL_TCOLS - _CK) * 128
        for tr in range(8):
            pltpu.async_copy(
                wt_hbm.at[pl.ds(8 * tr, 8), pl.ds(cdma, _CW)],
                wbuf.at[pl.ds(rb + 8 * tr, 8)], dsem)

    def fire_tail(k):
        rb = (k % 2) * _D
        for tr in range(8):
            pltpu.async_copy(
                wtail_hbm.at[pl.ds(8 * tr, 8)],
                wbuf.at[pl.ds(rb + 8 * tr, 8)], dsem)

    fire_full(0)
    pltpu.sync_copy(idx_hbm, idx_v)

    def scan_body(g, cnt):
        v = idx_v[pl.ds(g * 16, 16)]
        m = (v >= v_lo) & (v < v_hi)
        plsc.store_compressed(
            pk_v.at[pl.ds(cnt, 16)], (v << 14) | (g * 16 + lanes), mask=m)
        return cnt + plsc.all_reduce_population_count(m)[0]

    n_w = lax.fori_loop(0, _SEQ // 16, scan_body, jnp.int32(0), unroll=4)
    ngrp = (n_w + 15) // 16

    def chunk_body(k, wg):
        rb = (k % 2) * _D
        for tr in range(8):
            pltpu.make_async_copy(
                wt_hbm.at[pl.ds(0, 8), pl.ds(0, _CW)],
                wbuf.at[pl.ds(rb + 8 * tr, 8)], dsem).wait()
        nk = k + 1
        pl.when(nk < _NFULL)(lambda: fire_full(nk))
        pl.when(nk == _NFULL)(lambda: fire_tail(nk))

        c0 = jnp.where(k == _NFULL, _FULL_TCOLS, c0_w + _CK * k)
        ck = jnp.where(k == _NFULL,
                       (wid == _NW - 1).astype(jnp.int32),
                       jnp.clip(ncols - _CK * k, 0, _CK))
        lim = c0 * 128

        def rescan(g, ns):
            pk = pk_v[pl.ds(g * 16, 16)]
            o = (pk >> 14) - lim
            m = (lanes < n_w - g * 16) & (o >= 0) & (o < ck * 128)
            plsc.store_compressed(
                sub_v.at[pl.ds(ns, 16)], (o << 14) | (pk & 16383), mask=m)
            return ns + plsc.all_reduce_population_count(m)[0]

        n_sub = lax.fori_loop(0, ngrp, rescan, jnp.int32(0))
        row_m = [lanes + m * 16 + rb for m in range(4)]

        def window(wi, wg):
            def drain6():
                for _ in range(_WG):
                    pltpu.make_async_copy(
                        out_hbm.at[pl.ds(_TRASH, 16)],
                        rows_v.at[pl.ds(0, 16)], ssem).wait()
            pl.when(wg >= 2)(drain6)
            rbase = (wg % 2) * (_WG * 16)
            base = wi * (_WG * 16)
            for j in range(_WG):
                gbase = base + j * 16
                spk = sub_v[pl.ds(gbase, 16)]
                oc = (spk >> 14) & (_CW - 1)
                r = spk & 16383
                valid = lanes < n_sub - gbase
                srow = rbase + j * 16
                for jj in range(16):
                    oj = jnp.broadcast_to(oc[jj], (16,))
                    for m in range(4):
                        vals = plsc.load_gather(wbuf, [row_m[m], oj])
                        rows_v[srow + jj, pl.ds(m * 16, 16)] = vals
                rpad = jnp.where(valid, r, _TRASH)
                pltpu.async_copy(
                    rows_v.at[pl.ds(srow, 16)], out_hbm.at[rpad], ssem)
            return wg + 1

        nwin = (n_sub + _WG * 16 - 1) // (_WG * 16)
        return lax.fori_loop(0, nwin, window, wg)

    wg = lax.fori_loop(0, _NFULL + 1, chunk_body, jnp.int32(0))

    def drain(_, x):
        pltpu.make_async_copy(
            out_hbm.at[pl.ds(_TRASH, 16)], rows_v.at[pl.ds(0, 16)], ssem
        ).wait()
        return x

    lax.fori_loop(0, _WG * jnp.minimum(wg, 2), drain, 0)


def kernel(inputs, W):
    idx = inputs.astype(jnp.int32)
    wtail = jnp.pad(W[_TAIL_BASE:], ((0, _CW - (_V - _TAIL_BASE)), (0, 0))).T
    g = _emb(idx, W.T, wtail)
    return g[:_SEQ, :_D]


# speedup vs baseline: 1.8454x; 1.8454x over previous
import functools

import jax
import jax.numpy as jnp
from jax import lax
from jax.experimental import pallas as pl
from jax.experimental.pallas import tpu as pltpu
from jax.experimental.pallas import tpu_sc as plsc

_V = 100000
_D = 64
_SEQ = 16384
_NC, _NS = 2, 16
_NW = _NC * _NS
_FULL_TCOLS = 781
_TAIL_BASE = _FULL_TCOLS * 128
_CK = 4
_NFULL = 7
_TRASH = _SEQ
_OUT_ROWS = _SEQ + 128

_mesh = plsc.VectorSubcoreMesh(core_axis_name="c", subcore_axis_name="s")


@functools.partial(
    pl.kernel,
    mesh=_mesh,
    out_type=jax.ShapeDtypeStruct((_OUT_ROWS, 128), jnp.float32),
    scratch_types=[
        pltpu.VMEM((_SEQ,), jnp.int32),
        pltpu.VMEM((_SEQ,), jnp.int32),
        pltpu.VMEM((2 * _D, _CK * 128), jnp.float32),
        pltpu.VMEM((128, 128), jnp.float32),
        pltpu.SemaphoreType.DMA,
        pltpu.SemaphoreType.DMA,
    ],
    compiler_params=pltpu.CompilerParams(needs_layout_passes=False),
)
def _emb(idx_hbm, wt_hbm, wtail_hbm, out_hbm,
         sub_v, pk_v, wbuf, rows_v, dsem, ssem):
    idx_v = sub_v
    wid = lax.axis_index("s") * _NC + lax.axis_index("c")
    lanes = lax.iota(jnp.int32, 16)
    c0_w = 24 * wid + jnp.minimum(wid, 13)
    ncols = 24 + (wid < 13).astype(jnp.int32)
    v_lo = c0_w * 128
    v_hi = jnp.where(wid == _NW - 1, _V, (c0_w + ncols) * 128)

    def fire_full(k):
        rb = (k % 2) * _D
        cdma = jnp.minimum(c0_w + _CK * k, _FULL_TCOLS - _CK) * 128
        for tr in range(8):
            pltpu.async_copy(
                wt_hbm.at[pl.ds(8 * tr, 8), pl.ds(cdma, _CK * 128)],
                wbuf.at[pl.ds(rb + 8 * tr, 8)], dsem)

    def fire_tail(k):
        rb = (k % 2) * _D
        for tr in range(8):
            pltpu.async_copy(
                wtail_hbm.at[pl.ds(8 * tr, 8)],
                wbuf.at[pl.ds(rb + 8 * tr, 8)], dsem)

    fire_full(0)
    pltpu.sync_copy(idx_hbm, idx_v)

    def scan_body(g, cnt):
        v = idx_v[pl.ds(g * 16, 16)]
        m = (v >= v_lo) & (v < v_hi)
        plsc.store_compressed(
            pk_v.at[pl.ds(cnt, 16)], (v << 14) | (g * 16 + lanes), mask=m)
        return cnt + plsc.all_reduce_population_count(m)[0]

    n_w = lax.fori_loop(0, _SEQ // 16, scan_body, jnp.int32(0), unroll=4)
    ngrp = (n_w + 15) // 16

    def process_chunk(k, T):
        rb = (k % 2) * _D
        for tr in range(8):
            pltpu.make_async_copy(
                wt_hbm.at[pl.ds(0, 8), pl.ds(0, _CK * 128)],
                wbuf.at[pl.ds(rb + 8 * tr, 8)], dsem).wait()
        nk = k + 1
        pl.when(nk < _NFULL)(lambda: fire_full(nk))
        pl.when(nk == _NFULL)(lambda: fire_tail(nk))

        c0 = jnp.where(k == _NFULL, _FULL_TCOLS, c0_w + _CK * k)
        ck = jnp.where(k == _NFULL,
                       (wid == _NW - 1).astype(jnp.int32),
                       jnp.clip(ncols - _CK * k, 0, _CK))
        lim = c0 * 128

        def rescan(g, ns):
            pk = pk_v[pl.ds(g * 16, 16)]
            o = (pk >> 14) - lim
            m = (lanes < n_w - g * 16) & (o >= 0) & (o < ck * 128)
            plsc.store_compressed(
                sub_v.at[pl.ds(ns, 16)], (o << 14) | (pk & 16383), mask=m)
            return ns + plsc.all_reduce_population_count(m)[0]

        n_sub = lax.fori_loop(0, ngrp, rescan, jnp.int32(0))

        def dense(g, t):
            pl.when(t >= 8)(lambda: pltpu.make_async_copy(
                out_hbm.at[pl.ds(_TRASH, 16)], rows_v.at[pl.ds(0, 16)], ssem
            ).wait())
            slot = (t % 8) * 16
            spk = sub_v[pl.ds(g * 16, 16)]
            oc = (spk >> 14) & (_CK * 128 - 1)
            r = spk & 16383
            valid = lanes < n_sub - g * 16
            for j in range(16):
                oj = jnp.broadcast_to(oc[j], (16,))
                for m in range(4):
                    vals = plsc.load_gather(wbuf, [rb + lanes + m * 16, oj])
                    rows_v[slot + j, pl.ds(m * 16, 16)] = vals
            rpad = jnp.where(valid, r, _TRASH)
            pltpu.async_copy(rows_v.at[pl.ds(slot, 16)], out_hbm.at[rpad], ssem)
            return t + 1

        return lax.fori_loop(0, (n_sub + 15) // 16, dense, T)

    T = lax.fori_loop(0, _NFULL + 1, process_chunk, jnp.int32(0))

    def drain(_, x):
        pltpu.make_async_copy(
            out_hbm.at[pl.ds(_TRASH, 16)], rows_v.at[pl.ds(0, 16)], ssem
        ).wait()
        return x

    lax.fori_loop(0, jnp.minimum(T, 8), drain, 0)


def kernel(inputs, W):
    idx = inputs.astype(jnp.int32)
    wtail = jnp.pad(
        W[_TAIL_BASE:], ((0, _CK * 128 - (_V - _TAIL_BASE)), (0, 0))).T
    g = _emb(idx, W.T, wtail)
    return g[:_SEQ, :_D]


# speedup vs baseline: 2.3142x; 1.2540x over previous
import functools

import jax
import jax.numpy as jnp
from jax import lax
from jax.experimental import pallas as pl
from jax.experimental.pallas import tpu as pltpu
from jax.experimental.pallas import tpu_sc as plsc

_V = 100000
_D = 64
_SEQ = 16384
_NC, _NS = 2, 16
_NW = _NC * _NS
_FULL_TCOLS = 781
_TAIL_BASE = _FULL_TCOLS * 128
_CK = 8
_NCHUNK = 4
_TRASH = _SEQ
_OUT_ROWS = _SEQ + 128

_mesh = plsc.VectorSubcoreMesh(core_axis_name="c", subcore_axis_name="s")


@functools.partial(
    pl.kernel,
    mesh=_mesh,
    out_type=jax.ShapeDtypeStruct((_OUT_ROWS, 128), jnp.float32),
    scratch_types=[
        pltpu.VMEM((_SEQ,), jnp.int32),
        pltpu.VMEM((_SEQ,), jnp.int32),
        pltpu.VMEM((_D, _CK * 128), jnp.float32),
        pltpu.VMEM((128, 128), jnp.float32),
        pltpu.SemaphoreType.DMA,
        pltpu.SemaphoreType.DMA,
    ],
    compiler_params=pltpu.CompilerParams(needs_layout_passes=False),
)
def _emb(idx_hbm, wt_hbm, wtail_hbm, out_hbm,
         sub_v, pk_v, wbuf, rows_v, dsem, ssem):
    idx_v = sub_v
    wid = lax.axis_index("s") * _NC + lax.axis_index("c")
    lanes = lax.iota(jnp.int32, 16)
    c0_w = 24 * wid + jnp.minimum(wid, 13)
    ncols = 24 + (wid < 13).astype(jnp.int32)
    v_lo = c0_w * 128
    v_hi = jnp.where(wid == _NW - 1, _V, (c0_w + ncols) * 128)

    pltpu.sync_copy(idx_hbm, idx_v)

    def scan_body(g, cnt):
        v = idx_v[pl.ds(g * 16, 16)]
        m = (v >= v_lo) & (v < v_hi)
        plsc.store_compressed(
            pk_v.at[pl.ds(cnt, 16)], (v << 14) | (g * 16 + lanes), mask=m)
        return cnt + plsc.all_reduce_population_count(m)[0]

    n_w = lax.fori_loop(0, _SEQ // 16, scan_body, jnp.int32(0), unroll=4)
    ngrp = (n_w + 15) // 16

    def process_chunk(T, c0, ck, tail):
        fired = []
        cdma = jnp.minimum(c0, _FULL_TCOLS - _CK) * 128
        for tr in range(8):
            if tail:
                src = wtail_hbm.at[pl.ds(8 * tr, 8), pl.ds(0, 128)]
                dst = wbuf.at[pl.ds(8 * tr, 8), pl.ds(0, 128)]
            else:
                src = wt_hbm.at[pl.ds(8 * tr, 8), pl.ds(cdma, _CK * 128)]
                dst = wbuf.at[pl.ds(8 * tr, 8)]
            fired.append(pltpu.async_copy(src, dst, dsem))
        for cp in fired:
            cp.wait()

        lim = c0 * 128

        def rescan(g, ns):
            pk = pk_v[pl.ds(g * 16, 16)]
            o = (pk >> 14) - lim
            m = (lanes < n_w - g * 16) & (o >= 0) & (o < ck * 128)
            plsc.store_compressed(
                sub_v.at[pl.ds(ns, 16)], (o << 14) | (pk & 16383), mask=m)
            return ns + plsc.all_reduce_population_count(m)[0]

        n_sub = lax.fori_loop(0, ngrp, rescan, jnp.int32(0))

        def dense(g, t):
            pl.when(t >= 8)(lambda: pltpu.make_async_copy(
                out_hbm.at[pl.ds(_TRASH, 16)], rows_v.at[pl.ds(0, 16)], ssem
            ).wait())
            slot = (t % 8) * 16
            spk = sub_v[pl.ds(g * 16, 16)]
            r = spk & 16383
            valid = lanes < n_sub - g * 16
            for j in range(16):
                spkj = plsc.load_gather(
                    sub_v, [jnp.broadcast_to(g * 16 + j, (16,))])
                oj = (spkj >> 14) & (_CK * 128 - 1)
                for m in range(4):
                    vals = plsc.load_gather(wbuf, [lanes + m * 16, oj])
                    rows_v[slot + j, pl.ds(m * 16, 16)] = vals
            rpad = jnp.where(valid, r, _TRASH)
            pltpu.async_copy(rows_v.at[pl.ds(slot, 16)], out_hbm.at[rpad], ssem)
            return t + 1

        return lax.fori_loop(0, (n_sub + 15) // 16, dense, T)

    def chunk_body(k, T):
        c0 = c0_w + _CK * k
        ck = jnp.clip(ncols - _CK * k, 0, _CK)
        return process_chunk(T, c0, ck, tail=False)

    T = lax.fori_loop(0, _NCHUNK, chunk_body, jnp.int32(0))
    T = process_chunk(T, jnp.int32(_FULL_TCOLS),
                      jnp.where(wid == _NW - 1, 1, 0), tail=True)

    def drain(_, x):
        pltpu.make_async_copy(
            out_hbm.at[pl.ds(_TRASH, 16)], rows_v.at[pl.ds(0, 16)], ssem
        ).wait()
        return x

    lax.fori_loop(0, jnp.minimum(T, 8), drain, 0)


def kernel(inputs, W):
    idx = inputs.astype(jnp.int32)
    wtail = jnp.pad(W[_TAIL_BASE:], ((0, 128 - (_V - _TAIL_BASE)), (0, 0))).T
    g = _emb(idx, W.T, wtail)
    return g[:_SEQ, :_D]


# speedup vs baseline: 2.5391x; 1.0972x over previous
import functools

import jax
import jax.numpy as jnp
from jax import lax
from jax.experimental import pallas as pl
from jax.experimental.pallas import tpu as pltpu
from jax.experimental.pallas import tpu_sc as plsc

_VOCAB = 100000
_DIM = 64
_SEQ = 16384
_NC, _NS = 2, 16
_NW = _NC * _NS
_CHUNK = 128
_ROWS_PER_W = _SEQ // _NW
_CHUNKS_PER_W = _ROWS_PER_W // _CHUNK

_mesh = plsc.VectorSubcoreMesh(core_axis_name="c", subcore_axis_name="s")


@functools.partial(
    pl.kernel,
    mesh=_mesh,
    out_type=jax.ShapeDtypeStruct((_SEQ, _DIM), jnp.float32),
    scratch_types=[
        pltpu.VMEM((_ROWS_PER_W,), jnp.int32),
        pltpu.VMEM((_ROWS_PER_W, _DIM), jnp.float32),
        pltpu.SemaphoreType.DMA,
        pltpu.SemaphoreType.DMA,
    ],
    compiler_params=pltpu.CompilerParams(use_tc_tiling_on_sc=False),
)
def _gather_kernel(idx_hbm, table_hbm, out_hbm, idx_v, rows_v, g_sem, w_sem):
    wid = lax.axis_index("s") * _NC + lax.axis_index("c")
    base = wid * _ROWS_PER_W
    pltpu.sync_copy(idx_hbm.at[pl.ds(base, _ROWS_PER_W)], idx_v)
    gathers = []
    for j in range(_CHUNKS_PER_W):
        gathers.append(
            pltpu.async_copy(
                table_hbm.at[idx_v.at[pl.ds(j * _CHUNK, _CHUNK)]],
                rows_v.at[pl.ds(j * _CHUNK, _CHUNK)],
                g_sem,
            )
        )
    writes = []
    for j in range(_CHUNKS_PER_W):
        gathers[j].wait()
        writes.append(
            pltpu.async_copy(
                rows_v.at[pl.ds(j * _CHUNK, _CHUNK)],
                out_hbm.at[pl.ds(base + j * _CHUNK, _CHUNK)],
                w_sem,
            )
        )
    for c in writes:
        c.wait()


def kernel(inputs, W):
    return _gather_kernel(inputs.astype(jnp.int32), W)
